# Initial kernel scaffold; baseline (speedup 1.0000x reference)
import jax
import jax.numpy as jnp
from jax import lax
from jax.experimental import pallas as pl
from jax.experimental.pallas import tpu as pltpu
from jax.experimental.pallas import tpu_sc as plsc

N = 10000
NFEAT = 128
NHID = 64
NCLASS = 16
P = 160000

BM = 400
NBLK = N // BM

NC, NS = 2, 16
NW = NC * NS
PPW = P // NW
CHUNK = 1000
NCHUNK = PPW // CHUNK
IDX_ROWS, IDX_COLS = 8, 126
PAD = IDX_ROWS * IDX_COLS
NGRP = PAD // 16


def _gcn_body(adj_ref, x_ref, w1_ref, b1_ref, w2_ref, b2_ref, h2_ref,
              s1_scr, s2_scr):
    ph = pl.program_id(0)
    i = pl.program_id(1)

    @pl.when(jnp.logical_and(ph == 0, i == 0))
    def _():
        s1_scr[...] = jnp.dot(x_ref[...], w1_ref[...],
                              preferred_element_type=jnp.float32)

    @pl.when(ph == 0)
    def _():
        h = jnp.dot(adj_ref[...], s1_scr[...],
                    preferred_element_type=jnp.float32)
        h = jnp.maximum(h + b1_ref[...], 0.0)
        s2_scr[pl.ds(i * BM, BM), :] = jnp.dot(
            h, w2_ref[...], preferred_element_type=jnp.float32)

    @pl.when(ph == 1)
    def _():
        h2_ref[...] = jnp.dot(adj_ref[...], s2_scr[...],
                              preferred_element_type=jnp.float32) + b2_ref[...]


def _gcn(x, adj, W1, b1, W2, b2, interpret=False):
    return pl.pallas_call(
        _gcn_body,
        grid=(2, NBLK),
        in_specs=[
            pl.BlockSpec((BM, N), lambda p, i: (i, 0)),
            pl.BlockSpec((N, NFEAT), lambda p, i: (0, 0)),
            pl.BlockSpec((NFEAT, NHID), lambda p, i: (0, 0)),
            pl.BlockSpec((1, NHID), lambda p, i: (0, 0)),
            pl.BlockSpec((NHID, NCLASS), lambda p, i: (0, 0)),
            pl.BlockSpec((1, NCLASS), lambda p, i: (0, 0)),
        ],
        out_specs=pl.BlockSpec((BM, NCLASS), lambda p, i: (i, 0)),
        out_shape=jax.ShapeDtypeStruct((N, NCLASS), jnp.float32),
        scratch_shapes=[
            pltpu.VMEM((N, NHID), jnp.float32),
            pltpu.VMEM((N, NCLASS), jnp.float32),
        ],
        interpret=interpret,
    )(adj, x, W1, b1.reshape(1, NHID), W2, b2.reshape(1, NCLASS))


def _decode_body(h2_hbm, n1_hbm, n2_hbm, nit_hbm, out_hbm,
                 nit_v, idx1_v, idx2_v, map1_v, map2_v,
                 rows1_v, rows2_v, out_v, sem):
    wid = lax.axis_index("s") * NC + lax.axis_index("c")
    pltpu.sync_copy(nit_hbm, nit_v)
    lane = lax.iota(jnp.int32, 16)

    for c in range(NCHUNK):
        base = pl.multiple_of(wid * PPW + c * CHUNK, 8)
        pltpu.sync_copy(n1_hbm.at[pl.ds(base, CHUNK)], idx1_v)
        pltpu.sync_copy(n2_hbm.at[pl.ds(base, CHUNK)], idx2_v)

        def remap(g, carry):
            p = g * 16 + lane
            src = jnp.minimum(p, CHUNK - 1)
            r = p // IDX_COLS
            q = p - r * IDX_COLS
            ids1 = jnp.clip(plsc.load_gather(idx1_v, [src]), 0, N - 1)
            m1 = jnp.clip(plsc.load_gather(nit_v, [ids1]), 0, N - 1)
            plsc.store_scatter(map1_v, [r, q], m1)
            ids2 = jnp.clip(plsc.load_gather(idx2_v, [src]), 0, N - 1)
            m2 = jnp.clip(plsc.load_gather(nit_v, [ids2]), 0, N - 1)
            plsc.store_scatter(map2_v, [r, q], m2)
            return carry

        lax.fori_loop(0, NGRP, remap, 0)

        copies = []
        for j in range(IDX_ROWS):
            copies.append(pltpu.async_copy(
                h2_hbm.at[map1_v.at[j]],
                rows1_v.at[pl.ds(j * IDX_COLS, IDX_COLS)], sem))
            copies.append(pltpu.async_copy(
                h2_hbm.at[map2_v.at[j]],
                rows2_v.at[pl.ds(j * IDX_COLS, IDX_COLS)], sem))
        for cp in copies:
            cp.wait()

        def dot_group(g, carry):
            p = g * 16 + lane
            acc = jnp.zeros((16,), jnp.float32)
            for d in range(NCLASS):
                col = jnp.full((16,), d, jnp.int32)
                a = plsc.load_gather(rows1_v, [p, col])
                b = plsc.load_gather(rows2_v, [p, col])
                acc = acc + a * b
            plsc.store_scatter(out_v, [p], acc)
            return carry

        lax.fori_loop(0, NGRP, dot_group, 0)

        pltpu.sync_copy(out_v.at[pl.ds(0, CHUNK)], out_hbm.at[pl.ds(base, CHUNK)])


def _make_decode(interpret=False):
    return pl.kernel(
        _decode_body,
        out_type=jax.ShapeDtypeStruct((P,), jnp.float32),
        mesh=plsc.VectorSubcoreMesh(core_axis_name="c", subcore_axis_name="s",
                                    num_cores=NC, num_subcores=NS),
        scratch_types=[
            pltpu.VMEM((N,), jnp.int32),
            pltpu.VMEM((CHUNK,), jnp.int32),
            pltpu.VMEM((CHUNK,), jnp.int32),
            pltpu.VMEM((IDX_ROWS, IDX_COLS), jnp.int32),
            pltpu.VMEM((IDX_ROWS, IDX_COLS), jnp.int32),
            pltpu.VMEM((PAD, NCLASS), jnp.float32),
            pltpu.VMEM((PAD, NCLASS), jnp.float32),
            pltpu.VMEM((PAD,), jnp.float32),
            pltpu.SemaphoreType.DMA,
        ],
        interpret=interpret,
    )


@jax.jit
def kernel(x, adj, n1, n2, node_ids_to_index, W1, b1, W2, b2):
    h2 = _gcn(x, adj, W1, b1, W2, b2)
    decode = _make_decode()
    return decode(h2, n1.astype(jnp.int32), n2.astype(jnp.int32),
                  node_ids_to_index.astype(jnp.int32))


# speedup vs baseline: 11.1845x; 11.1845x over previous
import jax
import jax.numpy as jnp
from jax import lax
from jax.experimental import pallas as pl
from jax.experimental.pallas import tpu as pltpu
from jax.experimental.pallas import tpu_sc as plsc

N = 10000
NFEAT = 128
NHID = 64
NCLASS = 16
NPACK = NCLASS // 2
P = 160000

BM = 400
NBLK = N // BM

NC, NS = 2, 16
NW = NC * NS
PPW = P // NW
NGRP = -(-PPW // 16)
PADW = NGRP * 16


def _gcn_body(adj_ref, x_ref, w1_ref, b1_ref, w2_ref, b2_ref, h2_ref,
              s1_scr, s2_scr):
    ph = pl.program_id(0)
    i = pl.program_id(1)

    @pl.when(jnp.logical_and(ph == 0, i == 0))
    def _():
        s1_scr[...] = jnp.dot(x_ref[...], w1_ref[...],
                              preferred_element_type=jnp.float32)

    @pl.when(ph == 0)
    def _():
        h = jnp.dot(adj_ref[...], s1_scr[...],
                    preferred_element_type=jnp.float32)
        h = jnp.maximum(h + b1_ref[...], 0.0)
        s2_scr[pl.ds(i * BM, BM), :] = jnp.dot(
            h, w2_ref[...], preferred_element_type=jnp.float32)

    @pl.when(ph == 1)
    def _():
        h2_ref[...] = jnp.dot(adj_ref[...], s2_scr[...],
                              preferred_element_type=jnp.float32) + b2_ref[...]


def _gcn(x, adj, W1, b1, W2, b2, interpret=False):
    return pl.pallas_call(
        _gcn_body,
        grid=(2, NBLK),
        in_specs=[
            pl.BlockSpec((BM, N), lambda p, i: (i, 0)),
            pl.BlockSpec((N, NFEAT), lambda p, i: (0, 0)),
            pl.BlockSpec((NFEAT, NHID), lambda p, i: (0, 0)),
            pl.BlockSpec((1, NHID), lambda p, i: (0, 0)),
            pl.BlockSpec((NHID, NCLASS), lambda p, i: (0, 0)),
            pl.BlockSpec((1, NCLASS), lambda p, i: (0, 0)),
        ],
        out_specs=pl.BlockSpec((BM, NCLASS), lambda p, i: (i, 0)),
        out_shape=jax.ShapeDtypeStruct((N, NCLASS), jnp.float32),
        scratch_shapes=[
            pltpu.VMEM((N, NHID), jnp.float32),
            pltpu.VMEM((N, NCLASS), jnp.float32),
        ],
        interpret=interpret,
    )(adj, x, W1, b1.reshape(1, NHID), W2, b2.reshape(1, NCLASS))


def _decode_body(tab_hbm, n1_hbm, n2_hbm, nit_hbm, out_hbm,
                 tab_v, nit_v, idx1_v, idx2_v, out_v):
    wid = lax.axis_index("s") * NC + lax.axis_index("c")
    base = pl.multiple_of(wid * PPW, 8)
    pltpu.sync_copy(tab_hbm, tab_v)
    pltpu.sync_copy(nit_hbm, nit_v)
    pltpu.sync_copy(n1_hbm.at[pl.ds(base, PPW)], idx1_v)
    pltpu.sync_copy(n2_hbm.at[pl.ds(base, PPW)], idx2_v)

    @plsc.parallel_loop(0, NGRP)
    def dot_group(g):
        lane = lax.iota(jnp.int32, 16)
        p = g * 16 + lane
        src = jnp.minimum(p, PPW - 1)
        i1 = jnp.clip(plsc.load_gather(idx1_v, [src]), 0, N - 1)
        mi1 = jnp.clip(plsc.load_gather(nit_v, [i1]), 0, N - 1)
        i2 = jnp.clip(plsc.load_gather(idx2_v, [src]), 0, N - 1)
        mi2 = jnp.clip(plsc.load_gather(nit_v, [i2]), 0, N - 1)
        acc = jnp.zeros((16,), jnp.float32)
        w1 = mi1 * NPACK
        w2 = mi2 * NPACK
        for dd in range(NPACK):
            g1 = plsc.load_gather(tab_v, [w1 + dd])
            g2 = plsc.load_gather(tab_v, [w2 + dd])
            a1, b1 = plsc.unpack(plsc.bitcast(g1, jnp.bfloat16),
                                 format=plsc.PackFormat.INTERLEAVED,
                                 preferred_element_type=jnp.float32)
            a2, b2 = plsc.unpack(plsc.bitcast(g2, jnp.bfloat16),
                                 format=plsc.PackFormat.INTERLEAVED,
                                 preferred_element_type=jnp.float32)
            acc = acc + a1 * a2 + b1 * b2
        plsc.store_scatter(out_v, [p], acc)

    pltpu.sync_copy(out_v.at[pl.ds(0, PPW)], out_hbm.at[pl.ds(base, PPW)])


def _make_decode(interpret=False):
    return pl.kernel(
        _decode_body,
        out_type=jax.ShapeDtypeStruct((P,), jnp.float32),
        mesh=plsc.VectorSubcoreMesh(core_axis_name="c", subcore_axis_name="s",
                                    num_cores=NC, num_subcores=NS),
        compiler_params=pltpu.CompilerParams(needs_layout_passes=False),
        scratch_types=[
            pltpu.VMEM((N * NPACK,), jnp.int32),
            pltpu.VMEM((N,), jnp.int32),
            pltpu.VMEM((PPW,), jnp.int32),
            pltpu.VMEM((PPW,), jnp.int32),
            pltpu.VMEM((PADW,), jnp.float32),
        ],
        interpret=interpret,
    )


@jax.jit
def kernel(x, adj, n1, n2, node_ids_to_index, W1, b1, W2, b2):
    h2 = _gcn(x, adj, W1, b1, W2, b2)
    tab = lax.bitcast_convert_type(
        h2.astype(jnp.bfloat16).reshape(N, NPACK, 2), jnp.int32).reshape(-1)
    decode = _make_decode()
    return decode(tab, n1.astype(jnp.int32), n2.astype(jnp.int32),
                  node_ids_to_index.astype(jnp.int32))


# speedup vs baseline: 11.2034x; 1.0017x over previous
import jax
import jax.numpy as jnp
from jax import lax
from jax.experimental import pallas as pl
from jax.experimental.pallas import tpu as pltpu
from jax.experimental.pallas import tpu_sc as plsc

N = 10000
NFEAT = 128
NHID = 64
NCLASS = 16
NPACK = NCLASS // 2
P = 160000

BM = 400
NBLK = N // BM

NC, NS = 2, 16
NW = NC * NS
PPW = P // NW
NGRP = -(-PPW // 16)
PADW = NGRP * 16


def _gcn_body(adj_ref, x_ref, w1_ref, b1_ref, w2_ref, b2_ref, h2_ref,
              s1_scr, s2_scr):
    ph = pl.program_id(0)
    i = pl.program_id(1)

    @pl.when(jnp.logical_and(ph == 0, i == 0))
    def _():
        s1_scr[...] = jnp.dot(x_ref[...], w1_ref[...],
                              preferred_element_type=jnp.float32)

    @pl.when(ph == 0)
    def _():
        h = jnp.dot(adj_ref[...], s1_scr[...],
                    preferred_element_type=jnp.float32)
        h = jnp.maximum(h + b1_ref[...], 0.0)
        s2_scr[pl.ds(i * BM, BM), :] = jnp.dot(
            h, w2_ref[...], preferred_element_type=jnp.float32)

    @pl.when(ph == 1)
    def _():
        h2_ref[...] = jnp.dot(adj_ref[...], s2_scr[...],
                              preferred_element_type=jnp.float32) + b2_ref[...]


def _gcn(x, adj, W1, b1, W2, b2, interpret=False):
    return pl.pallas_call(
        _gcn_body,
        grid=(2, NBLK),
        in_specs=[
            pl.BlockSpec((BM, N), lambda p, i: (i, 0)),
            pl.BlockSpec((N, NFEAT), lambda p, i: (0, 0)),
            pl.BlockSpec((NFEAT, NHID), lambda p, i: (0, 0)),
            pl.BlockSpec((1, NHID), lambda p, i: (0, 0)),
            pl.BlockSpec((NHID, NCLASS), lambda p, i: (0, 0)),
            pl.BlockSpec((1, NCLASS), lambda p, i: (0, 0)),
        ],
        out_specs=pl.BlockSpec((BM, NCLASS), lambda p, i: (i, 0)),
        out_shape=jax.ShapeDtypeStruct((N, NCLASS), jnp.float32),
        scratch_shapes=[
            pltpu.VMEM((N, NHID), jnp.float32),
            pltpu.VMEM((N, NCLASS), jnp.float32),
        ],
        interpret=interpret,
    )(adj, x, W1, b1.reshape(1, NHID), W2, b2.reshape(1, NCLASS))


def _decode_body(tab_hbm, n1_hbm, n2_hbm, nit_hbm, out_hbm,
                 tab_v, nit_v, idx1_v, idx2_v, out_v, sem):
    wid = lax.axis_index("s") * NC + lax.axis_index("c")
    base = pl.multiple_of(wid * PPW, 8)
    lane = lax.iota(jnp.int32, 16)
    copies = [
        pltpu.async_copy(tab_hbm, tab_v, sem),
        pltpu.async_copy(nit_hbm, nit_v, sem),
        pltpu.async_copy(n1_hbm.at[pl.ds(base, PPW)],
                         idx1_v.at[pl.ds(0, PPW)], sem),
        pltpu.async_copy(n2_hbm.at[pl.ds(base, PPW)],
                         idx2_v.at[pl.ds(0, PPW)], sem),
    ]
    for cp in copies:
        cp.wait()
    zeros16 = jnp.zeros((16,), jnp.int32)
    plsc.store_scatter(idx1_v, [PPW + lane], zeros16)
    plsc.store_scatter(idx2_v, [PPW + lane], zeros16)

    @plsc.parallel_loop(0, NGRP, unroll=4)
    def dot_group(g):
        l16 = lax.iota(jnp.int32, 16)
        p = g * 16 + l16
        i1 = idx1_v[pl.ds(g * 16, 16)]
        mi1 = plsc.load_gather(nit_v, [i1])
        i2 = idx2_v[pl.ds(g * 16, 16)]
        mi2 = plsc.load_gather(nit_v, [i2])
        acc = jnp.zeros((16,), jnp.float32)
        w1 = mi1 * NPACK
        w2 = mi2 * NPACK
        for dd in range(NPACK):
            g1 = plsc.load_gather(tab_v, [w1 + dd])
            g2 = plsc.load_gather(tab_v, [w2 + dd])
            a1, b1 = plsc.unpack(plsc.bitcast(g1, jnp.bfloat16),
                                 format=plsc.PackFormat.INTERLEAVED,
                                 preferred_element_type=jnp.float32)
            a2, b2 = plsc.unpack(plsc.bitcast(g2, jnp.bfloat16),
                                 format=plsc.PackFormat.INTERLEAVED,
                                 preferred_element_type=jnp.float32)
            acc = acc + a1 * a2 + b1 * b2
        plsc.store_scatter(out_v, [p], acc)

    pltpu.sync_copy(out_v.at[pl.ds(0, PPW)], out_hbm.at[pl.ds(base, PPW)])


def _make_decode(interpret=False):
    return pl.kernel(
        _decode_body,
        out_type=jax.ShapeDtypeStruct((P,), jnp.float32),
        mesh=plsc.VectorSubcoreMesh(core_axis_name="c", subcore_axis_name="s",
                                    num_cores=NC, num_subcores=NS),
        compiler_params=pltpu.CompilerParams(needs_layout_passes=False),
        scratch_types=[
            pltpu.VMEM((N * NPACK,), jnp.int32),
            pltpu.VMEM((N,), jnp.int32),
            pltpu.VMEM((PADW + 16,), jnp.int32),
            pltpu.VMEM((PADW + 16,), jnp.int32),
            pltpu.VMEM((PADW,), jnp.float32),
            pltpu.SemaphoreType.DMA,
        ],
        interpret=interpret,
    )


@jax.jit
def kernel(x, adj, n1, n2, node_ids_to_index, W1, b1, W2, b2):
    h2 = _gcn(x, adj, W1, b1, W2, b2)
    tab = lax.bitcast_convert_type(
        h2.astype(jnp.bfloat16).reshape(N, NPACK, 2), jnp.int32).reshape(-1)
    decode = _make_decode()
    return decode(tab, n1.astype(jnp.int32), n2.astype(jnp.int32),
                  node_ids_to_index.astype(jnp.int32))


# speedup vs baseline: 11.4883x; 1.0254x over previous
import jax
import jax.numpy as jnp
from jax import lax
from jax.experimental import pallas as pl
from jax.experimental.pallas import tpu as pltpu
from jax.experimental.pallas import tpu_sc as plsc

N = 10000
NFEAT = 128
NHID = 64
NCLASS = 16
NPACK = NCLASS // 2
P = 160000

BM = 400
NBLK = N // BM

NC, NS = 2, 16
NW = NC * NS
PPW = P // NW
NGRP = -(-PPW // 16)
PADW = NGRP * 16


def _gcn_body(adj_ref, x_ref, w1_ref, b1_ref, w2_ref, b2_ref, tab_ref,
              s1_scr, s2_scr):
    ph = pl.program_id(0)
    i = pl.program_id(1)

    @pl.when(jnp.logical_and(ph == 0, i == 0))
    def _():
        s1_scr[...] = jnp.dot(x_ref[...], w1_ref[...],
                              preferred_element_type=jnp.float32)

    @pl.when(ph == 0)
    def _():
        h = jnp.dot(adj_ref[...], s1_scr[...],
                    preferred_element_type=jnp.float32)
        h = jnp.maximum(h + b1_ref[...], 0.0)
        s2_scr[pl.ds(i * BM, BM), :] = jnp.dot(
            h, w2_ref[...], preferred_element_type=jnp.float32)

    @pl.when(ph == 1)
    def _():
        h2 = jnp.dot(adj_ref[...], s2_scr[...],
                     preferred_element_type=jnp.float32) + b2_ref[...]
        bits = pltpu.bitcast(h2, jnp.int32)
        lsb = jax.lax.shift_right_logical(bits, 16) & 1
        b16 = jax.lax.shift_right_logical(bits + 0x7FFF + lsb, 16)
        high = pltpu.roll(b16, 8, axis=1)
        tab_ref[...] = (b16 | (high << 16))[:, 0:NPACK]


def _gcn(x, adj, W1, b1, W2, b2, interpret=False):
    return pl.pallas_call(
        _gcn_body,
        grid=(2, NBLK),
        in_specs=[
            pl.BlockSpec((BM, N), lambda p, i: (i, 0)),
            pl.BlockSpec((N, NFEAT), lambda p, i: (0, 0)),
            pl.BlockSpec((NFEAT, NHID), lambda p, i: (0, 0)),
            pl.BlockSpec((1, NHID), lambda p, i: (0, 0)),
            pl.BlockSpec((NHID, NCLASS), lambda p, i: (0, 0)),
            pl.BlockSpec((1, NCLASS), lambda p, i: (0, 0)),
        ],
        out_specs=pl.BlockSpec((BM, NPACK), lambda p, i: (i, 0)),
        out_shape=jax.ShapeDtypeStruct((N, NPACK), jnp.int32),
        scratch_shapes=[
            pltpu.VMEM((N, NHID), jnp.float32),
            pltpu.VMEM((N, NCLASS), jnp.float32),
        ],
        interpret=interpret,
    )(adj, x, W1, b1.reshape(1, NHID), W2, b2.reshape(1, NCLASS))


def _decode_body(tab_hbm, n1_hbm, n2_hbm, nit_hbm, out_hbm,
                 tab_v, nit_v, idx1_v, idx2_v, out_v, sem):
    wid = lax.axis_index("s") * NC + lax.axis_index("c")
    base = pl.multiple_of(wid * PPW, 8)
    lane = lax.iota(jnp.int32, 16)
    copies = [
        pltpu.async_copy(tab_hbm, tab_v, sem),
        pltpu.async_copy(nit_hbm, nit_v, sem),
        pltpu.async_copy(n1_hbm.at[pl.ds(base, PPW)],
                         idx1_v.at[pl.ds(0, PPW)], sem),
        pltpu.async_copy(n2_hbm.at[pl.ds(base, PPW)],
                         idx2_v.at[pl.ds(0, PPW)], sem),
    ]
    for cp in copies:
        cp.wait()
    zeros16 = jnp.zeros((16,), jnp.int32)
    plsc.store_scatter(idx1_v, [PPW + lane], zeros16)
    plsc.store_scatter(idx2_v, [PPW + lane], zeros16)

    @plsc.parallel_loop(0, NGRP, unroll=4)
    def dot_group(g):
        l16 = lax.iota(jnp.int32, 16)
        p = g * 16 + l16
        i1 = idx1_v[pl.ds(g * 16, 16)]
        mi1 = plsc.load_gather(nit_v, [i1])
        i2 = idx2_v[pl.ds(g * 16, 16)]
        mi2 = plsc.load_gather(nit_v, [i2])
        acc = jnp.zeros((16,), jnp.float32)
        w1 = mi1 * NPACK
        w2 = mi2 * NPACK
        for dd in range(NPACK):
            g1 = plsc.load_gather(tab_v, [w1 + dd])
            g2 = plsc.load_gather(tab_v, [w2 + dd])
            a1, b1 = plsc.unpack(plsc.bitcast(g1, jnp.bfloat16),
                                 format=plsc.PackFormat.INTERLEAVED,
                                 preferred_element_type=jnp.float32)
            a2, b2 = plsc.unpack(plsc.bitcast(g2, jnp.bfloat16),
                                 format=plsc.PackFormat.INTERLEAVED,
                                 preferred_element_type=jnp.float32)
            acc = acc + a1 * a2 + b1 * b2
        plsc.store_scatter(out_v, [p], acc)

    pltpu.sync_copy(out_v.at[pl.ds(0, PPW)], out_hbm.at[pl.ds(base, PPW)])


def _make_decode(interpret=False):
    return pl.kernel(
        _decode_body,
        out_type=jax.ShapeDtypeStruct((P,), jnp.float32),
        mesh=plsc.VectorSubcoreMesh(core_axis_name="c", subcore_axis_name="s",
                                    num_cores=NC, num_subcores=NS),
        compiler_params=pltpu.CompilerParams(needs_layout_passes=False),
        scratch_types=[
            pltpu.VMEM((N * NPACK,), jnp.int32),
            pltpu.VMEM((N,), jnp.int32),
            pltpu.VMEM((PADW + 16,), jnp.int32),
            pltpu.VMEM((PADW + 16,), jnp.int32),
            pltpu.VMEM((PADW,), jnp.float32),
            pltpu.SemaphoreType.DMA,
        ],
        interpret=interpret,
    )


@jax.jit
def kernel(x, adj, n1, n2, node_ids_to_index, W1, b1, W2, b2):
    tab = _gcn(x, adj, W1, b1, W2, b2).reshape(-1)
    decode = _make_decode()
    return decode(tab, n1.astype(jnp.int32), n2.astype(jnp.int32),
                  node_ids_to_index.astype(jnp.int32))
